# baseline (speedup 1.0000x reference)
import functools

import jax
import jax.numpy as jnp
from jax import lax
from jax.experimental import pallas as pl
from jax.experimental.pallas import tpu as pltpu
from jax.experimental.pallas import tpu_sc as plsc

_B = 16384
_D = 32
_NC = 2
_NS = 16
_NW = _NC * _NS
_BPW = _B // _NW


def _fpmc_body(uid_hbm, lid_hbm, pid_hbm, uit_hbm, iut_hbm, ilt_hbm, lit_hbm,
               out_hbm,
               idx_u, idx_l, idx_p, bu, bi, bm, bn, acc_v,
               sem0, sem1, sem2, sem3):
    wid = lax.axis_index("s") * _NC + lax.axis_index("c")
    base = wid * _BPW

    pltpu.sync_copy(uid_hbm.at[pl.ds(base, _BPW)], idx_u)
    pltpu.sync_copy(lid_hbm.at[pl.ds(base, _BPW)], idx_l)
    pltpu.sync_copy(pid_hbm.at[pl.ds(base, _BPW)], idx_p)

    for ch in range(_BPW // 16):
        acc_v[pl.ds(ch * 16, 16)] = jnp.zeros((16,), jnp.float32)

    def body(d, carry):
        cu = pltpu.async_copy(uit_hbm.at[d].at[idx_u], bu, sem0)
        ci = pltpu.async_copy(iut_hbm.at[d].at[idx_p], bi, sem1)
        cm = pltpu.async_copy(lit_hbm.at[d].at[idx_l], bm, sem2)
        cn = pltpu.async_copy(ilt_hbm.at[d].at[idx_p], bn, sem3)
        cu.wait()
        ci.wait()
        cm.wait()
        cn.wait()
        for ch in range(_BPW // 16):
            sl = pl.ds(ch * 16, 16)
            acc_v[sl] = acc_v[sl] + bu[sl] * bi[sl] + bm[sl] * bn[sl]
        return carry

    lax.fori_loop(0, _D, body, 0)
    pltpu.sync_copy(acc_v, out_hbm.at[pl.ds(base, _BPW)])


@jax.jit
def _fpmc(user_ids, last_items, pre_items, UIt, IUt, ILt, LIt):
    mesh = plsc.VectorSubcoreMesh(core_axis_name="c", subcore_axis_name="s")
    run = pl.kernel(
        _fpmc_body,
        out_type=jax.ShapeDtypeStruct((_B,), jnp.float32),
        mesh=mesh,
        compiler_params=pltpu.CompilerParams(
            needs_layout_passes=False, use_tc_tiling_on_sc=False),
        scratch_types=[
            pltpu.VMEM((_BPW,), jnp.int32),
            pltpu.VMEM((_BPW,), jnp.int32),
            pltpu.VMEM((_BPW,), jnp.int32),
            pltpu.VMEM((_BPW,), jnp.float32),
            pltpu.VMEM((_BPW,), jnp.float32),
            pltpu.VMEM((_BPW,), jnp.float32),
            pltpu.VMEM((_BPW,), jnp.float32),
            pltpu.VMEM((_BPW,), jnp.float32),
            pltpu.SemaphoreType.DMA,
            pltpu.SemaphoreType.DMA,
            pltpu.SemaphoreType.DMA,
            pltpu.SemaphoreType.DMA,
        ],
    )
    return run(user_ids, last_items, pre_items, UIt, IUt, ILt, LIt)


def kernel(user_ids, last_items, pre_items, UI, IU, IL, LI):
    return _fpmc(user_ids.astype(jnp.int32), last_items.astype(jnp.int32),
                 pre_items.astype(jnp.int32), UI.T, IU.T, IL.T, LI.T)


# speedup vs baseline: 5.6571x; 5.6571x over previous
import functools

import jax
import jax.numpy as jnp
from jax import lax
from jax.experimental import pallas as pl
from jax.experimental.pallas import tpu as pltpu
from jax.experimental.pallas import tpu_sc as plsc

_B = 16384
_D = 32
_DP = 128
_NC = 2
_NS = 16
_NW = _NC * _NS
_BPW = _B // _NW
_CH = 64


def _fpmc_body(uid_hbm, lid_hbm, pid_hbm, ui_hbm, iu_hbm, il_hbm, li_hbm,
               out_hbm,
               idx_u, idx_l, idx_p, ui_v, iu_v, il_v, li_v, tr_v, out_v,
               sem0, sem1, sem2, sem3):
    wid = lax.axis_index("s") * _NC + lax.axis_index("c")
    base = wid * _BPW

    pltpu.sync_copy(uid_hbm.at[pl.ds(base, _BPW)], idx_u)
    pltpu.sync_copy(lid_hbm.at[pl.ds(base, _BPW)], idx_l)
    pltpu.sync_copy(pid_hbm.at[pl.ds(base, _BPW)], idx_p)

    lane = lax.broadcasted_iota(jnp.int32, (16,), 0)

    def body(g, carry):
        cu = pltpu.async_copy(ui_hbm.at[idx_u.at[pl.ds(g * _CH, _CH)]], ui_v, sem0)
        ci = pltpu.async_copy(iu_hbm.at[idx_p.at[pl.ds(g * _CH, _CH)]], iu_v, sem1)
        cm = pltpu.async_copy(li_hbm.at[idx_l.at[pl.ds(g * _CH, _CH)]], li_v, sem2)
        cn = pltpu.async_copy(il_hbm.at[idx_p.at[pl.ds(g * _CH, _CH)]], il_v, sem3)
        cu.wait()
        ci.wait()
        cm.wait()
        cn.wait()
        for gg in range(_CH // 16):
            for j in range(16):
                b = gg * 16 + j
                p = ui_v[b, pl.ds(0, 16)] * iu_v[b, pl.ds(0, 16)]
                p = p + ui_v[b, pl.ds(16, 16)] * iu_v[b, pl.ds(16, 16)]
                p = p + li_v[b, pl.ds(0, 16)] * il_v[b, pl.ds(0, 16)]
                p = p + li_v[b, pl.ds(16, 16)] * il_v[b, pl.ds(16, 16)]
                plsc.store_scatter(tr_v, [lane * 16 + j], p)
            acc = tr_v[pl.ds(0, 16)]
            for i in range(1, 16):
                acc = acc + tr_v[pl.ds(i * 16, 16)]
            out_v[pl.ds(g * _CH + gg * 16, 16)] = acc
        return carry

    lax.fori_loop(0, _BPW // _CH, body, 0)
    pltpu.sync_copy(out_v, out_hbm.at[pl.ds(base, _BPW)])


@jax.jit
def _fpmc(user_ids, last_items, pre_items, UI, IU, IL, LI):
    UIp = jnp.pad(UI, ((0, 0), (0, _DP - _D)))
    IUp = jnp.pad(IU, ((0, 0), (0, _DP - _D)))
    ILp = jnp.pad(IL, ((0, 0), (0, _DP - _D)))
    LIp = jnp.pad(LI, ((0, 0), (0, _DP - _D)))
    mesh = plsc.VectorSubcoreMesh(core_axis_name="c", subcore_axis_name="s")
    run = pl.kernel(
        _fpmc_body,
        out_type=jax.ShapeDtypeStruct((_B,), jnp.float32),
        mesh=mesh,
        compiler_params=pltpu.CompilerParams(needs_layout_passes=False),
        scratch_types=[
            pltpu.VMEM((_BPW,), jnp.int32),
            pltpu.VMEM((_BPW,), jnp.int32),
            pltpu.VMEM((_BPW,), jnp.int32),
            pltpu.VMEM((_CH, _DP), jnp.float32),
            pltpu.VMEM((_CH, _DP), jnp.float32),
            pltpu.VMEM((_CH, _DP), jnp.float32),
            pltpu.VMEM((_CH, _DP), jnp.float32),
            pltpu.VMEM((256,), jnp.float32),
            pltpu.VMEM((_BPW,), jnp.float32),
            pltpu.SemaphoreType.DMA,
            pltpu.SemaphoreType.DMA,
            pltpu.SemaphoreType.DMA,
            pltpu.SemaphoreType.DMA,
        ],
    )
    return run(user_ids, last_items, pre_items, UIp, IUp, ILp, LIp)


def kernel(user_ids, last_items, pre_items, UI, IU, IL, LI):
    return _fpmc(user_ids.astype(jnp.int32), last_items.astype(jnp.int32),
                 pre_items.astype(jnp.int32), UI, IU, IL, LI)


# speedup vs baseline: 5.9207x; 1.0466x over previous
import functools

import jax
import jax.numpy as jnp
from jax import lax
from jax.experimental import pallas as pl
from jax.experimental.pallas import tpu as pltpu
from jax.experimental.pallas import tpu_sc as plsc

_B = 16384
_D = 32
_NC = 2
_NS = 16
_NW = _NC * _NS
_BPW = _B // _NW


def _fpmc_body(uid_hbm, lid_hbm, pid_hbm, ui_hbm, iu_hbm, il_hbm, li_hbm,
               out_hbm,
               idx_u, idx_l, idx_p, ui_v, iu_v, il_v, li_v, tr_v, out_v,
               sem0, sem1, sem2, sem3):
    wid = lax.axis_index("s") * _NC + lax.axis_index("c")
    base = wid * _BPW

    pltpu.sync_copy(uid_hbm.at[pl.ds(base, _BPW)], idx_u)
    pltpu.sync_copy(lid_hbm.at[pl.ds(base, _BPW)], idx_l)
    pltpu.sync_copy(pid_hbm.at[pl.ds(base, _BPW)], idx_p)

    cu = pltpu.async_copy(ui_hbm.at[idx_u], ui_v, sem0)
    ci = pltpu.async_copy(iu_hbm.at[idx_p], iu_v, sem1)
    cl = pltpu.async_copy(il_hbm.at[idx_p], il_v, sem2)
    cm = pltpu.async_copy(li_hbm.at[idx_l], li_v, sem3)
    cu.wait()
    ci.wait()
    cl.wait()
    cm.wait()

    lane = lax.broadcasted_iota(jnp.int32, (16,), 0)

    def body(g, carry):
        for j in range(16):
            b = g * 16 + j
            p = ui_v[b, pl.ds(0, 16)] * iu_v[b, pl.ds(0, 16)]
            p = p + ui_v[b, pl.ds(16, 16)] * iu_v[b, pl.ds(16, 16)]
            p = p + li_v[b, pl.ds(0, 16)] * il_v[b, pl.ds(0, 16)]
            p = p + li_v[b, pl.ds(16, 16)] * il_v[b, pl.ds(16, 16)]
            plsc.store_scatter(tr_v, [lane * 16 + j], p)
        acc = tr_v[pl.ds(0, 16)]
        for i in range(1, 16):
            acc = acc + tr_v[pl.ds(i * 16, 16)]
        out_v[pl.ds(g * 16, 16)] = acc
        return carry

    lax.fori_loop(0, _BPW // 16, body, 0)
    pltpu.sync_copy(out_v, out_hbm.at[pl.ds(base, _BPW)])


@jax.jit
def _fpmc(user_ids, last_items, pre_items, UI, IU, IL, LI):
    mesh = plsc.VectorSubcoreMesh(core_axis_name="c", subcore_axis_name="s")
    run = pl.kernel(
        _fpmc_body,
        out_type=jax.ShapeDtypeStruct((_B,), jnp.float32),
        mesh=mesh,
        compiler_params=pltpu.CompilerParams(
            needs_layout_passes=False, use_tc_tiling_on_sc=False),
        scratch_types=[
            pltpu.VMEM((_BPW,), jnp.int32),
            pltpu.VMEM((_BPW,), jnp.int32),
            pltpu.VMEM((_BPW,), jnp.int32),
            pltpu.VMEM((_BPW, _D), jnp.float32),
            pltpu.VMEM((_BPW, _D), jnp.float32),
            pltpu.VMEM((_BPW, _D), jnp.float32),
            pltpu.VMEM((_BPW, _D), jnp.float32),
            pltpu.VMEM((256,), jnp.float32),
            pltpu.VMEM((_BPW,), jnp.float32),
            pltpu.SemaphoreType.DMA,
            pltpu.SemaphoreType.DMA,
            pltpu.SemaphoreType.DMA,
            pltpu.SemaphoreType.DMA,
        ],
    )
    return run(user_ids, last_items, pre_items, UI, IU, IL, LI)


def kernel(user_ids, last_items, pre_items, UI, IU, IL, LI):
    return _fpmc(user_ids.astype(jnp.int32), last_items.astype(jnp.int32),
                 pre_items.astype(jnp.int32), UI, IU, IL, LI)
